# baseline (speedup 1.0000x reference)
import functools

import jax
import jax.numpy as jnp
from jax import lax
from jax.experimental import pallas as pl
from jax.experimental.pallas import tpu as pltpu
from jax.experimental.pallas import tpu_sc as plsc

NUM_USERS = 80000
NUM_ITEMS = 20000
DIM = 64
HOP = 3
NUM_NEG = 8
ALPHA = 0.5
N_NODES = NUM_USERS + NUM_ITEMS
NPAD = 100096
N_EDGES = 1000000
BATCH = 1024

G = 4
GD = 16
EPAD = 1 << 20
NS = 16
NC = 2
E_TILE = EPAD // NS
NB = 512
NBLK = E_TILE // NB
ROWS_TILE = NPAD // NS
ZROWS = 782

_mesh = plsc.VectorSubcoreMesh(core_axis_name="c", subcore_axis_name="s")


def _hop_body(x0_hbm, rows_hbm, cols_hbm, vals_hbm, x1_hbm, x2_hbm, x3_hbm,
              y_sh, colsv, rowsv, valsv, gbuf, semi, semg):
    c = lax.axis_index("c")
    s = lax.axis_index("s")
    zero16 = jnp.zeros((GD,), jnp.float32)
    base = s * ROWS_TILE
    NJ = NB // 128

    def idx_issue(blk, m):
        off = s * E_TILE + blk * NB
        for j in range(NJ):
            pltpu.async_copy(cols_hbm.at[pl.ds(off + j * 128, 128)],
                             colsv.at[m * NJ + j], semi)
            pltpu.async_copy(rows_hbm.at[pl.ds(off + j * 128, 128)],
                             rowsv.at[m * NJ + j], semi)
        pltpu.async_copy(vals_hbm.at[pl.ds(off, NB)],
                         valsv.at[pl.ds(m * NB, NB)], semi)

    def idx_drain(m):
        for j in range(NJ):
            pltpu.make_async_copy(cols_hbm.at[pl.ds(0, 128)],
                                  colsv.at[m * NJ + j], semi).wait()
            pltpu.make_async_copy(rows_hbm.at[pl.ds(0, 128)],
                                  rowsv.at[m * NJ + j], semi).wait()
        pltpu.make_async_copy(vals_hbm.at[pl.ds(0, NB)],
                              valsv.at[pl.ds(m * NB, NB)], semi).wait()

    def gather_issue(x_hbm, g, m, p):
        for j in range(NJ):
            pltpu.async_copy(x_hbm.at[g].at[colsv.at[m * NJ + j]],
                             gbuf.at[pl.ds(p * NB + j * 128, 128)], semg)

    def gather_drain(x_hbm, p):
        for j in range(NJ):
            pltpu.make_async_copy(x_hbm.at[0].at[pl.ds(0, 128)],
                                  gbuf.at[pl.ds(p * NB + j * 128, 128)],
                                  semg).wait()

    hop_io = [(x0_hbm, x1_hbm), (x1_hbm, x2_hbm), (x2_hbm, x3_hbm)]
    for x_hbm, out_hbm in hop_io:
      for g_local in range(2):
        g = c * 2 + g_local
        gN = g * NPAD
        @plsc.parallel_loop(0, 2 * NB, unroll=8)
        def _(i):
            gbuf[i, :] = zero16

        for z in range(ROWS_TILE // (2 * NB)):
            pltpu.sync_copy(gbuf, y_sh.at[pl.ds(base + z * 2 * NB, 2 * NB)])
        rem = ROWS_TILE % (2 * NB)
        if rem:
            pltpu.sync_copy(gbuf.at[pl.ds(0, rem)],
                            y_sh.at[pl.ds(base + ROWS_TILE - rem, rem)])
        plsc.subcore_barrier()

        idx_issue(0, 0)
        idx_drain(0)
        gather_issue(x_hbm, g, 0, 0)
        idx_issue(1, 1)

        def blk_body(t, _):
            m = t % 3
            mp = (t - 1) % 3
            p = t % 2
            q = 1 - p
            blk = t % NBLK
            idx_drain(m)
            gather_drain(x_hbm, q)
            gather_issue(x_hbm, g, m, p)
            idx_issue((t + 1) % NBLK, (t + 1) % 3)

            @plsc.parallel_loop(0, NB // 16, unroll=2)
            def _(i):
                e0 = i * 16
                vv = valsv[pl.ds(mp * NB + e0, 16)]
                for l in range(16):
                    gbuf[q * NB + e0 + l, :] = gbuf[q * NB + e0 + l, :] * vv[l]

            for j in range(NJ):
                pltpu.sync_copy(gbuf.at[pl.ds(q * NB + j * 128, 128)],
                                y_sh.at[rowsv.at[mp * NJ + j]], add=True)
            return ()

        lax.fori_loop(1, NBLK + 1, blk_body, (), unroll=False)
        gather_drain(x_hbm, 0)
        idx_drain((NBLK + 1) % 3)
        plsc.subcore_barrier()
        pltpu.sync_copy(y_sh.at[pl.ds(base, ROWS_TILE)],
                        out_hbm.at[g].at[pl.ds(base, ROWS_TILE)])
        plsc.subcore_barrier()


_hop = pl.kernel(
    _hop_body,
    out_type=(jax.ShapeDtypeStruct((G, NPAD, GD), jnp.float32),
              jax.ShapeDtypeStruct((G, NPAD, GD), jnp.float32),
              jax.ShapeDtypeStruct((G, NPAD, GD), jnp.float32)),
    mesh=_mesh,
    compiler_params=pltpu.CompilerParams(use_tc_tiling_on_sc=False),
    scratch_types=[
        pltpu.VMEM_SHARED((NPAD, GD), jnp.float32),
        pltpu.VMEM((3 * (NB // 128), 128), jnp.int32),
        pltpu.VMEM((3 * (NB // 128), 128), jnp.int32),
        pltpu.VMEM((3 * NB,), jnp.float32),
        pltpu.VMEM((2 * NB, GD), jnp.float32),
        pltpu.SemaphoreType.DMA,
        pltpu.SemaphoreType.DMA,
    ],
)


def _gather_body(x0_hbm, x1_hbm, x2_hbm, x3_hbm, prior_hbm,
                 users_hbm, items_hbm, cand_hbm,
                 u_out, i_out, c_out, p_out,
                 idxv, rbuf, abuf, pv, pidx, pbuf, sem):
    c = lax.axis_index("c")
    s = lax.axis_index("s")
    w = s * NC + c
    g = w // 8
    part = w % 8
    gN = g * NPAD
    xs = [x0_hbm, x1_hbm, x2_hbm, x3_hbm]

    def gather_rows(src2d_hbm, src_row, shift, dst, dst_off):
        pltpu.sync_copy(src2d_hbm.at[pl.ds(src_row, 1)], idxv)
        if shift:
            for q in range(8):
                idxv[0, pl.ds(q * 16, 16)] = idxv[0, pl.ds(q * 16, 16)] + shift
        cps = [pltpu.async_copy(x.at[g].at[idxv.at[0]], rbuf.at[h], sem)
               for h, x in enumerate(xs)]
        for cp in cps:
            cp.wait()

        @plsc.parallel_loop(0, 128, unroll=4)
        def _(i):
            abuf[i, :] = (rbuf[0, i, :] + rbuf[1, i, :]
                          + rbuf[2, i, :] + rbuf[3, i, :]) * 0.25

        pltpu.sync_copy(abuf, dst.at[pl.ds(dst_off, 128)])

    gather_rows(users_hbm, part, 0, u_out, g * BATCH + part * 128)
    gather_rows(items_hbm, part, NUM_USERS, i_out, g * BATCH + part * 128)
    for r in range(8):
        row = part * 8 + r
        gather_rows(cand_hbm, row, NUM_USERS, c_out,
                    g * (BATCH * NUM_NEG) + row * 128)
    pltpu.sync_copy(prior_hbm, pv)
    pltpu.sync_copy(cand_hbm.at[pl.ds(w * 2, 2)], pidx)
    for j in range(2):
        for q in range(8):
            ids = pidx[j, pl.ds(q * 16, 16)]
            pbuf[j, pl.ds(q * 16, 16)] = plsc.load_gather(pv, [ids])
    pltpu.sync_copy(pbuf, p_out.at[pl.ds(w * 2, 2)])


_gather = pl.kernel(
    _gather_body,
    out_type=(
        jax.ShapeDtypeStruct((G * BATCH, GD), jnp.float32),
        jax.ShapeDtypeStruct((G * BATCH, GD), jnp.float32),
        jax.ShapeDtypeStruct((G * BATCH * NUM_NEG, GD), jnp.float32),
        jax.ShapeDtypeStruct((BATCH * NUM_NEG // 128, 128), jnp.float32),
    ),
    mesh=_mesh,
    compiler_params=pltpu.CompilerParams(use_tc_tiling_on_sc=False,
                                          needs_layout_passes=False),
    scratch_types=[
        pltpu.VMEM((1, 128), jnp.int32),
        pltpu.VMEM((4, 128, GD), jnp.float32),
        pltpu.VMEM((128, GD), jnp.float32),
        pltpu.VMEM((NUM_ITEMS,), jnp.float32),
        pltpu.VMEM((2, 128), jnp.int32),
        pltpu.VMEM((2, 128), jnp.float32),
        pltpu.SemaphoreType.DMA,
    ],
)


def _candscore_body(u, i, cnd, xui, ct):
    uu = u[...]
    xui[...] = jnp.sum(uu * i[...], axis=(0, 2))[None, :]
    ct[...] = jnp.sum(uu[:, None, :, :] * cnd[...], axis=(0, 3))


def _candscore(u4, i4, c4):
    return pl.pallas_call(
        _candscore_body,
        out_shape=(jax.ShapeDtypeStruct((1, BATCH), jnp.float32),
                   jax.ShapeDtypeStruct((NUM_NEG, BATCH), jnp.float32)),
    )(u4, i4, c4)


def _count_body(l0, l1, l2, l3, u, ct, o):
    i = pl.program_id(0)
    s = jnp.zeros((BATCH, 2000), jnp.float32)
    for g in range(G):
        m = (l0[g] + l1[g] + l2[g] + l3[g]) * 0.25
        s = s + lax.dot_general(u[g], m, (((1,), (1,)), ((), ())),
                                preferred_element_type=jnp.float32)
    cnt = jnp.concatenate(
        [jnp.sum((s <= ct[k, :][:, None]).astype(jnp.float32), axis=1)[None, :]
         for k in range(NUM_NEG)], axis=0)

    @pl.when(i == 0)
    def _():
        o[...] = cnt

    @pl.when(i != 0)
    def _():
        o[...] = o[...] + cnt


def _count(xs3, u4, ct):
    lspec = pl.BlockSpec((G, 2000, GD), lambda i: (0, i + NUM_USERS // 2000, 0))
    return pl.pallas_call(
        _count_body,
        grid=(10,),
        in_specs=[lspec, lspec, lspec, lspec,
                  pl.BlockSpec((G, BATCH, GD), lambda i: (0, 0, 0)),
                  pl.BlockSpec((NUM_NEG, BATCH), lambda i: (0, 0))],
        out_specs=pl.BlockSpec((NUM_NEG, BATCH), lambda i: (0, 0)),
        out_shape=jax.ShapeDtypeStruct((NUM_NEG, BATCH), jnp.float32),
    )(*xs3, u4, ct)


def _final_body(cnt, ct, xui, pfn, o):
    f = cnt[...] / (NUM_ITEMS + 1)
    cs = ct[...]
    p = pfn[...]
    info = 1.0 - jax.nn.sigmoid(xui[...] - cs)
    unbias = (1.0 - f) * (1.0 - p) / (1.0 - f - p + 2.0 * f * p)
    risk = info * (1.0 - (1.0 + ALPHA) * unbias)
    best = risk[0, :]
    bestsc = cs[0, :]
    for k in range(1, NUM_NEG):
        lt = risk[k, :] < best
        bestsc = jnp.where(lt, cs[k, :], bestsc)
        best = jnp.where(lt, risk[k, :], best)
    z = bestsc - xui[0, :]
    o[...] = jnp.mean(jnp.log1p(jnp.exp(-jnp.abs(z))) + jnp.maximum(z, 0.0)).reshape(1, 1)


def _final(cnt, ct, xui, pfn):
    return pl.pallas_call(
        _final_body,
        out_shape=jax.ShapeDtypeStruct((1, 1), jnp.float32),
    )(cnt, ct, xui, pfn)


def kernel(user_emb, item_emb, A_vals, prior, A_rows, A_cols, users, items,
           candidates, epoch):
    all_emb = jnp.concatenate([
        user_emb, item_emb,
        jnp.zeros((NPAD - N_NODES, DIM), jnp.float32)], axis=0)
    x0 = all_emb.reshape(NPAD, G, GD).transpose(1, 0, 2)

    npad = EPAD - N_EDGES
    padr = (jnp.arange(npad, dtype=jnp.int32) * 17) % N_NODES
    rows32 = jnp.concatenate([A_rows.astype(jnp.int32), padr])
    cols32 = jnp.concatenate([A_cols.astype(jnp.int32), padr])
    vals = jnp.concatenate([A_vals, jnp.zeros((npad,), jnp.float32)])

    x1, x2, x3 = _hop(x0, rows32, cols32, vals)

    users2d = users.astype(jnp.int32).reshape(8, 128)
    items2d = items.astype(jnp.int32).reshape(8, 128)
    cand2d = candidates.astype(jnp.int32).T.reshape(NUM_NEG * BATCH // 128, 128)

    u_f, i_f, c_f, p_f = _gather(x0, x1, x2, x3, prior, users2d, items2d, cand2d)
    u4 = u_f.reshape(G, BATCH, GD)
    i4 = i_f.reshape(G, BATCH, GD)
    c4 = c_f.reshape(G, NUM_NEG, BATCH, GD)
    pfn = p_f.reshape(NUM_NEG, BATCH)

    xui, ct = _candscore(u4, i4, c4)
    cnt = _count([x0, x1, x2, x3], u4, ct)
    loss = _final(cnt, ct, xui, pfn)
    return loss.reshape(())


# speedup vs baseline: 1.2012x; 1.2012x over previous
import functools

import jax
import jax.numpy as jnp
from jax import lax
from jax.experimental import pallas as pl
from jax.experimental.pallas import tpu as pltpu
from jax.experimental.pallas import tpu_sc as plsc

NUM_USERS = 80000
NUM_ITEMS = 20000
DIM = 64
HOP = 3
NUM_NEG = 8
ALPHA = 0.5
N_NODES = NUM_USERS + NUM_ITEMS
NPAD = 100096
N_EDGES = 1000000
BATCH = 1024

G = 4
GD = 16
EPAD = 1 << 20
NS = 16
NC = 2
E_TILE = EPAD // NS
NB = 512
NBLK = E_TILE // NB
ROWS_TILE = NPAD // NS
ZROWS = 782

_mesh = plsc.VectorSubcoreMesh(core_axis_name="c", subcore_axis_name="s")


def _hop_body(x0_hbm, rows_hbm, cols_hbm, vals_hbm, x1_hbm, x2_hbm, x3_hbm,
              y_sh, colsv, rowsv, valsv, gbuf, semi, semg):
    c = lax.axis_index("c")
    s = lax.axis_index("s")
    zero16 = jnp.zeros((GD,), jnp.float32)
    base = s * ROWS_TILE
    NJ = NB // 128

    def idx_issue(blk, m):
        off = s * E_TILE + blk * NB
        for j in range(NJ):
            pltpu.async_copy(cols_hbm.at[pl.ds(off + j * 128, 128)],
                             colsv.at[m * NJ + j], semi)
            pltpu.async_copy(rows_hbm.at[pl.ds(off + j * 128, 128)],
                             rowsv.at[m * NJ + j], semi)
        pltpu.async_copy(vals_hbm.at[pl.ds(off, NB)],
                         valsv.at[pl.ds(m * NB, NB)], semi)

    def idx_drain(m):
        for j in range(NJ):
            pltpu.make_async_copy(cols_hbm.at[pl.ds(0, 128)],
                                  colsv.at[m * NJ + j], semi).wait()
            pltpu.make_async_copy(rows_hbm.at[pl.ds(0, 128)],
                                  rowsv.at[m * NJ + j], semi).wait()
        pltpu.make_async_copy(vals_hbm.at[pl.ds(0, NB)],
                              valsv.at[pl.ds(m * NB, NB)], semi).wait()

    def gather_issue(x_hbm, g, m, p):
        for j in range(NJ):
            pltpu.async_copy(x_hbm.at[g].at[colsv.at[m * NJ + j]],
                             gbuf.at[pl.ds(p * NB + j * 128, 128)], semg)

    def gather_drain(x_hbm, p):
        for j in range(NJ):
            pltpu.make_async_copy(x_hbm.at[0].at[pl.ds(0, 128)],
                                  gbuf.at[pl.ds(p * NB + j * 128, 128)],
                                  semg).wait()

    hop_io = [(x0_hbm, x1_hbm), (x1_hbm, x2_hbm), (x2_hbm, x3_hbm)]
    for x_hbm, out_hbm in hop_io:
      for g_local in range(2):
        g = c * 2 + g_local
        gN = g * NPAD
        @plsc.parallel_loop(0, 2 * NB, unroll=8)
        def _(i):
            gbuf[i, :] = zero16

        for z in range(ROWS_TILE // (2 * NB)):
            pltpu.sync_copy(gbuf, y_sh.at[pl.ds(base + z * 2 * NB, 2 * NB)])
        rem = ROWS_TILE % (2 * NB)
        if rem:
            pltpu.sync_copy(gbuf.at[pl.ds(0, rem)],
                            y_sh.at[pl.ds(base + ROWS_TILE - rem, rem)])
        plsc.subcore_barrier()

        idx_issue(0, 0)
        idx_drain(0)
        gather_issue(x_hbm, g, 0, 0)
        idx_issue(1, 1)

        def blk_body(t, _):
            m = t % 3
            mp = (t - 1) % 3
            p = t % 2
            q = 1 - p
            blk = t % NBLK
            idx_drain(m)
            gather_drain(x_hbm, q)
            gather_issue(x_hbm, g, m, p)
            idx_issue((t + 1) % NBLK, (t + 1) % 3)

            @plsc.parallel_loop(0, NB // 16, unroll=2)
            def _(i):
                e0 = i * 16
                vv = valsv[pl.ds(mp * NB + e0, 16)]
                for l in range(16):
                    gbuf[q * NB + e0 + l, :] = gbuf[q * NB + e0 + l, :] * vv[l]

            for j in range(NJ):
                pltpu.sync_copy(gbuf.at[pl.ds(q * NB + j * 128, 128)],
                                y_sh.at[rowsv.at[mp * NJ + j]], add=True)
            return ()

        lax.fori_loop(1, NBLK + 1, blk_body, (), unroll=False)
        gather_drain(x_hbm, 0)
        idx_drain((NBLK + 1) % 3)
        plsc.subcore_barrier()
        pltpu.sync_copy(y_sh.at[pl.ds(base, ROWS_TILE)],
                        out_hbm.at[g].at[pl.ds(base, ROWS_TILE)])
        plsc.subcore_barrier()


_hop = pl.kernel(
    _hop_body,
    out_type=(jax.ShapeDtypeStruct((G, NPAD, GD), jnp.float32),
              jax.ShapeDtypeStruct((G, NPAD, GD), jnp.float32),
              jax.ShapeDtypeStruct((G, NPAD, GD), jnp.float32)),
    mesh=_mesh,
    compiler_params=pltpu.CompilerParams(use_tc_tiling_on_sc=False),
    scratch_types=[
        pltpu.VMEM_SHARED((NPAD, GD), jnp.float32),
        pltpu.VMEM((3 * (NB // 128), 128), jnp.int32),
        pltpu.VMEM((3 * (NB // 128), 128), jnp.int32),
        pltpu.VMEM((3 * NB,), jnp.float32),
        pltpu.VMEM((2 * NB, GD), jnp.float32),
        pltpu.SemaphoreType.DMA,
        pltpu.SemaphoreType.DMA,
    ],
)


def _gather_body(x0_hbm, x1_hbm, x2_hbm, x3_hbm, prior_hbm,
                 users_hbm, items_hbm, cand_hbm,
                 u_out, i_out, c_out, p_out, m_out,
                 idxv, rbuf, abuf, mbuf, mabuf, pv, pidx, pbuf, sem):
    c = lax.axis_index("c")
    s = lax.axis_index("s")
    w = s * NC + c
    g = w // 8
    part = w % 8
    gN = g * NPAD
    xs = [x0_hbm, x1_hbm, x2_hbm, x3_hbm]

    def gather_rows(src2d_hbm, src_row, shift, dst, dst_off):
        pltpu.sync_copy(src2d_hbm.at[pl.ds(src_row, 1)], idxv)
        if shift:
            for q in range(8):
                idxv[0, pl.ds(q * 16, 16)] = idxv[0, pl.ds(q * 16, 16)] + shift
        cps = [pltpu.async_copy(x.at[g].at[idxv.at[0]], rbuf.at[h], sem)
               for h, x in enumerate(xs)]
        for cp in cps:
            cp.wait()

        @plsc.parallel_loop(0, 128, unroll=4)
        def _(i):
            abuf[i, :] = (rbuf[0, i, :] + rbuf[1, i, :]
                          + rbuf[2, i, :] + rbuf[3, i, :]) * 0.25

        pltpu.sync_copy(abuf, dst.at[pl.ds(dst_off, 128)])

    gather_rows(users_hbm, part, 0, u_out, g * BATCH + part * 128)
    gather_rows(items_hbm, part, NUM_USERS, i_out, g * BATCH + part * 128)
    for r in range(8):
        row = part * 8 + r
        gather_rows(cand_hbm, row, NUM_USERS, c_out,
                    g * (BATCH * NUM_NEG) + row * 128)
    for ch in range(5):
        off = NUM_USERS + part * 2500 + ch * 500
        for h, x in enumerate(xs):
            pltpu.sync_copy(x.at[g].at[pl.ds(off, 500)], mbuf.at[h])

        @plsc.parallel_loop(0, 500, unroll=4)
        def _(i):
            mabuf[i, :] = (mbuf[0, i, :] + mbuf[1, i, :]
                           + mbuf[2, i, :] + mbuf[3, i, :]) * 0.25

        pltpu.sync_copy(mabuf, m_out.at[g].at[pl.ds(part * 2500 + ch * 500, 500)])

    pltpu.sync_copy(prior_hbm, pv)
    pltpu.sync_copy(cand_hbm.at[pl.ds(w * 2, 2)], pidx)
    for j in range(2):
        for q in range(8):
            ids = pidx[j, pl.ds(q * 16, 16)]
            pbuf[j, pl.ds(q * 16, 16)] = plsc.load_gather(pv, [ids])
    pltpu.sync_copy(pbuf, p_out.at[pl.ds(w * 2, 2)])


_gather = pl.kernel(
    _gather_body,
    out_type=(
        jax.ShapeDtypeStruct((G * BATCH, GD), jnp.float32),
        jax.ShapeDtypeStruct((G * BATCH, GD), jnp.float32),
        jax.ShapeDtypeStruct((G * BATCH * NUM_NEG, GD), jnp.float32),
        jax.ShapeDtypeStruct((BATCH * NUM_NEG // 128, 128), jnp.float32),
        jax.ShapeDtypeStruct((G, NUM_ITEMS, GD), jnp.float32),
    ),
    mesh=_mesh,
    compiler_params=pltpu.CompilerParams(use_tc_tiling_on_sc=False,
                                          needs_layout_passes=False),
    scratch_types=[
        pltpu.VMEM((1, 128), jnp.int32),
        pltpu.VMEM((4, 128, GD), jnp.float32),
        pltpu.VMEM((128, GD), jnp.float32),
        pltpu.VMEM((4, 500, GD), jnp.float32),
        pltpu.VMEM((500, GD), jnp.float32),
        pltpu.VMEM((NUM_ITEMS,), jnp.float32),
        pltpu.VMEM((2, 128), jnp.int32),
        pltpu.VMEM((2, 128), jnp.float32),
        pltpu.SemaphoreType.DMA,
    ],
)


def _candscore_body(u, i, cnd, xui, ct):
    uu = u[...]
    xui[...] = jnp.sum(uu * i[...], axis=(0, 2))[None, :]
    ct[...] = jnp.sum(uu[:, None, :, :] * cnd[...], axis=(0, 3))


def _candscore(u4, i4, c4):
    return pl.pallas_call(
        _candscore_body,
        out_shape=(jax.ShapeDtypeStruct((1, BATCH), jnp.float32),
                   jax.ShapeDtypeStruct((NUM_NEG, BATCH), jnp.float32)),
    )(u4, i4, c4)


def _count_body(li, u, ct, o):
    i = pl.program_id(0)
    s = jnp.zeros((BATCH, 2000), jnp.float32)
    for g in range(G):
        s = s + lax.dot_general(u[g], li[g], (((1,), (1,)), ((), ())),
                                preferred_element_type=jnp.float32)
    cnt = jnp.concatenate(
        [jnp.sum((s <= ct[k, :][:, None]).astype(jnp.float32), axis=1)[None, :]
         for k in range(NUM_NEG)], axis=0)

    @pl.when(i == 0)
    def _():
        o[...] = cnt

    @pl.when(i != 0)
    def _():
        o[...] = o[...] + cnt


def _count(m3, u4, ct):
    return pl.pallas_call(
        _count_body,
        grid=(10,),
        in_specs=[pl.BlockSpec((G, 2000, GD), lambda i: (0, i, 0)),
                  pl.BlockSpec((G, BATCH, GD), lambda i: (0, 0, 0)),
                  pl.BlockSpec((NUM_NEG, BATCH), lambda i: (0, 0))],
        out_specs=pl.BlockSpec((NUM_NEG, BATCH), lambda i: (0, 0)),
        out_shape=jax.ShapeDtypeStruct((NUM_NEG, BATCH), jnp.float32),
    )(m3, u4, ct)


def _final_body(cnt, ct, xui, pfn, o):
    f = cnt[...] / (NUM_ITEMS + 1)
    cs = ct[...]
    p = pfn[...]
    info = 1.0 - jax.nn.sigmoid(xui[...] - cs)
    unbias = (1.0 - f) * (1.0 - p) / (1.0 - f - p + 2.0 * f * p)
    risk = info * (1.0 - (1.0 + ALPHA) * unbias)
    best = risk[0, :]
    bestsc = cs[0, :]
    for k in range(1, NUM_NEG):
        lt = risk[k, :] < best
        bestsc = jnp.where(lt, cs[k, :], bestsc)
        best = jnp.where(lt, risk[k, :], best)
    z = bestsc - xui[0, :]
    o[...] = jnp.mean(jnp.log1p(jnp.exp(-jnp.abs(z))) + jnp.maximum(z, 0.0)).reshape(1, 1)


def _final(cnt, ct, xui, pfn):
    return pl.pallas_call(
        _final_body,
        out_shape=jax.ShapeDtypeStruct((1, 1), jnp.float32),
    )(cnt, ct, xui, pfn)


def kernel(user_emb, item_emb, A_vals, prior, A_rows, A_cols, users, items,
           candidates, epoch):
    all_emb = jnp.concatenate([
        user_emb, item_emb,
        jnp.zeros((NPAD - N_NODES, DIM), jnp.float32)], axis=0)
    x0 = all_emb.reshape(NPAD, G, GD).transpose(1, 0, 2)

    npad = EPAD - N_EDGES
    padr = (jnp.arange(npad, dtype=jnp.int32) * 17) % N_NODES
    rows32 = jnp.concatenate([A_rows.astype(jnp.int32), padr])
    cols32 = jnp.concatenate([A_cols.astype(jnp.int32), padr])
    vals = jnp.concatenate([A_vals, jnp.zeros((npad,), jnp.float32)])

    x1, x2, x3 = _hop(x0, rows32, cols32, vals)

    users2d = users.astype(jnp.int32).reshape(8, 128)
    items2d = items.astype(jnp.int32).reshape(8, 128)
    cand2d = candidates.astype(jnp.int32).T.reshape(NUM_NEG * BATCH // 128, 128)

    u_f, i_f, c_f, p_f, m3 = _gather(x0, x1, x2, x3, prior, users2d, items2d,
                                     cand2d)
    u4 = u_f.reshape(G, BATCH, GD)
    i4 = i_f.reshape(G, BATCH, GD)
    c4 = c_f.reshape(G, NUM_NEG, BATCH, GD)
    pfn = p_f.reshape(NUM_NEG, BATCH)

    xui, ct = _candscore(u4, i4, c4)
    cnt = _count(m3, u4, ct)
    loss = _final(cnt, ct, xui, pfn)
    return loss.reshape(())
